# baseline (speedup 1.0000x reference)
import functools

import jax
import jax.numpy as jnp
from jax import lax
from jax.experimental import pallas as pl
from jax.experimental.pallas import tpu as pltpu
from jax.experimental.pallas import tpu_sc as plsc

_LANES = 16
_BLK = 128


@functools.cache
def _build(b: int, t: int, d: int):
    total = b * t
    info = plsc.get_sparse_core_info()
    n_workers = info.num_cores * info.num_subcores
    per_w = total // n_workers
    b_per_w = b // n_workers
    n_chunks = per_w // _BLK
    mesh = plsc.VectorSubcoreMesh(core_axis_name="c", subcore_axis_name="s")

    @functools.partial(
        pl.kernel,
        mesh=mesh,
        out_type=jax.ShapeDtypeStruct((b, t, d), jnp.float32),
        scratch_types=[
            pltpu.VMEM((per_w,), jnp.int32),
            pltpu.VMEM((per_w, d), jnp.float32),
            pltpu.SemaphoreType.DMA,
            pltpu.SemaphoreType.DMA,
        ],
        compiler_params=pltpu.CompilerParams(use_tc_tiling_on_sc=False),
    )
    def gather_kernel(idx_hbm, cache_hbm, out_hbm, idx_v, rows_v, sem, sem_o):
        wid = lax.axis_index("s") * info.num_cores + lax.axis_index("c")
        base = wid * per_w
        b_base = wid * b_per_w
        pltpu.sync_copy(idx_hbm.at[pl.ds(base, per_w)], idx_v)

        def row_body(i, carry):
            t_off = (i % jnp.int32(t)) * jnp.int32(d)
            for v in range(_BLK // _LANES):
                sl = pl.ds(i * _BLK + v * _LANES, _LANES)
                idx_v[sl] = idx_v[sl] + t_off
            return carry

        lax.fori_loop(0, n_chunks, row_body, 0)

        copies = []
        for j in range(n_chunks):
            sl = pl.ds(j * _BLK, _BLK)
            copies.append(
                pltpu.async_copy(cache_hbm.at[idx_v.at[sl]], rows_v.at[sl], sem)
            )
        outs = []
        for j in range(n_chunks):
            copies[j].wait()
            outs.append(
                pltpu.async_copy(
                    rows_v.at[pl.ds(j * _BLK, _BLK)],
                    out_hbm.at[pl.ds(b_base + (j // t) * _BLK, _BLK), j % t],
                    sem_o,
                )
            )
        for o in outs:
            o.wait()

    return gather_kernel


def kernel(idx, outputs):
    b, t = idx.shape
    d = outputs.shape[1]
    cache = jnp.concatenate(
        [
            lax.slice(outputs, (0, 0), ((d - 1) * d**p + 1, d), (d**p, 1))
            for p in range(t)
        ],
        axis=0,
    )
    idx_blocks = idx.reshape(b // _BLK, _BLK, t).transpose(0, 2, 1).reshape(-1)
    return _build(b, t, d)(idx_blocks, cache)


# speedup vs baseline: 1.4856x; 1.4856x over previous
import functools

import jax
import jax.numpy as jnp
from jax import lax
from jax.experimental import pallas as pl
from jax.experimental.pallas import tpu as pltpu
from jax.experimental.pallas import tpu_sc as plsc

_LANES = 16
_BLK = 128


@functools.cache
def _build(b: int, t: int, d: int):
    total = b * t
    info = plsc.get_sparse_core_info()
    n_workers = info.num_cores * info.num_subcores
    per_w = total // n_workers
    b_per_w = b // n_workers
    n_chunks = per_w // _BLK
    mesh = plsc.VectorSubcoreMesh(core_axis_name="c", subcore_axis_name="s")

    @functools.partial(
        pl.kernel,
        mesh=mesh,
        out_type=jax.ShapeDtypeStruct((b, t, d), jnp.float32),
        scratch_types=[
            pltpu.VMEM((per_w,), jnp.int32),
            pltpu.VMEM((per_w, d), jnp.float32),
            pltpu.SemaphoreType.DMA,
            pltpu.SemaphoreType.DMA,
        ],
        compiler_params=pltpu.CompilerParams(use_tc_tiling_on_sc=False),
    )
    def gather_kernel(idx_hbm, cache_hbm, out_hbm, idx_v, rows_v, sem, sem_o):
        wid = lax.axis_index("s") * info.num_cores + lax.axis_index("c")
        base = wid * per_w
        b_base = wid * b_per_w
        pltpu.sync_copy(idx_hbm.at[pl.ds(base, per_w)], idx_v)

        def row_body(i, carry):
            t_off = (i % jnp.int32(t)) * jnp.int32(d)
            for v in range(_BLK // _LANES):
                sl = pl.ds(i * _BLK + v * _LANES, _LANES)
                idx_v[sl] = idx_v[sl] + t_off
            return carry

        lax.fori_loop(0, n_chunks, row_body, 0)

        copies = []
        for j in range(n_chunks):
            sl = pl.ds(j * _BLK, _BLK)
            copies.append(
                pltpu.async_copy(cache_hbm.at[idx_v.at[sl]], rows_v.at[sl], sem)
            )
        outs = []
        for j in range(n_chunks):
            copies[j].wait()
            outs.append(
                pltpu.async_copy(
                    rows_v.at[pl.ds(j * _BLK, _BLK)],
                    out_hbm.at[pl.ds(b_base + (j // t) * _BLK, _BLK), j % t],
                    sem_o,
                )
            )
        for o in outs:
            o.wait()

    return gather_kernel


def kernel(idx, outputs):
    b, t = idx.shape
    d = outputs.shape[1]
    parts = []
    for p in range(t):
        stride = d**p
        if stride <= d * d:
            parts.append(
                lax.slice(outputs, (0, 0), ((d - 1) * stride + 1, d), (stride, 1))
            )
        else:
            parts.extend(
                lax.slice(outputs, (v * stride, 0), (v * stride + 1, d))
                for v in range(d)
            )
    cache = jnp.concatenate(parts, axis=0)
    idx_blocks = idx.reshape(b // _BLK, _BLK, t).transpose(0, 2, 1).reshape(-1)
    return _build(b, t, d)(idx_blocks, cache)


# speedup vs baseline: 1.8376x; 1.2370x over previous
import functools

import jax
import jax.numpy as jnp
from jax import lax
from jax.experimental import pallas as pl
from jax.experimental.pallas import tpu as pltpu
from jax.experimental.pallas import tpu_sc as plsc

_LANES = 16
_BLK = 128
_REPS = 8


@functools.cache
def _build(b: int, t: int, d: int):
    total = b * t
    info = plsc.get_sparse_core_info()
    n_workers = info.num_cores * info.num_subcores
    per_w = total // n_workers
    b_per_w = b // n_workers
    n_chunks = per_w // _BLK
    mesh = plsc.VectorSubcoreMesh(core_axis_name="c", subcore_axis_name="s")

    @functools.partial(
        pl.kernel,
        mesh=mesh,
        out_type=jax.ShapeDtypeStruct((b, t, d), jnp.float32),
        scratch_types=[
            pltpu.VMEM((per_w,), jnp.int32),
            pltpu.VMEM((per_w, d), jnp.float32),
            pltpu.SemaphoreType.DMA,
            pltpu.SemaphoreType.DMA,
        ],
        compiler_params=pltpu.CompilerParams(use_tc_tiling_on_sc=False),
    )
    def gather_kernel(idx_hbm, cache_hbm, out_hbm, idx_v, rows_v, sem, sem_o):
        wid = lax.axis_index("s") * info.num_cores + lax.axis_index("c")
        base = wid * per_w
        b_base = wid * b_per_w
        pltpu.sync_copy(idx_hbm.at[pl.ds(base, per_w)], idx_v)

        rep_off = (wid % jnp.int32(_REPS)) * jnp.int32(t * d)

        def row_body(i, carry):
            t_off = rep_off + (i % jnp.int32(t)) * jnp.int32(d)
            for v in range(_BLK // _LANES):
                sl = pl.ds(i * _BLK + v * _LANES, _LANES)
                idx_v[sl] = idx_v[sl] + t_off
            return carry

        lax.fori_loop(0, n_chunks, row_body, 0)

        copies = []
        for j in range(n_chunks):
            sl = pl.ds(j * _BLK, _BLK)
            copies.append(
                pltpu.async_copy(cache_hbm.at[idx_v.at[sl]], rows_v.at[sl], sem)
            )
        outs = []
        for j in range(n_chunks):
            copies[j].wait()
            outs.append(
                pltpu.async_copy(
                    rows_v.at[pl.ds(j * _BLK, _BLK)],
                    out_hbm.at[pl.ds(b_base + (j // t) * _BLK, _BLK), j % t],
                    sem_o,
                )
            )
        for o in outs:
            o.wait()

    return gather_kernel


def kernel(idx, outputs):
    b, t = idx.shape
    d = outputs.shape[1]
    parts = []
    for p in range(t):
        stride = d**p
        if stride <= d * d:
            parts.append(
                lax.slice(outputs, (0, 0), ((d - 1) * stride + 1, d), (stride, 1))
            )
        else:
            parts.extend(
                lax.slice(outputs, (v * stride, 0), (v * stride + 1, d))
                for v in range(d)
            )
    cache = jnp.tile(jnp.concatenate(parts, axis=0), (_REPS, 1))
    idx_blocks = idx.reshape(b // _BLK, _BLK, t).transpose(0, 2, 1).reshape(-1)
    return _build(b, t, d)(idx_blocks, cache)


# speedup vs baseline: 2.0200x; 1.0992x over previous
import functools

import jax
import jax.numpy as jnp
from jax import lax
from jax.experimental import pallas as pl
from jax.experimental.pallas import tpu as pltpu
from jax.experimental.pallas import tpu_sc as plsc

_LANES = 16
_BLK = 128
_REPS = 32


@functools.cache
def _build(b: int, t: int, d: int):
    total = b * t
    info = plsc.get_sparse_core_info()
    n_workers = info.num_cores * info.num_subcores
    per_w = total // n_workers
    b_per_w = b // n_workers
    n_chunks = per_w // _BLK
    mesh = plsc.VectorSubcoreMesh(core_axis_name="c", subcore_axis_name="s")

    @functools.partial(
        pl.kernel,
        mesh=mesh,
        out_type=jax.ShapeDtypeStruct((b, t, d), jnp.float32),
        scratch_types=[
            pltpu.VMEM((per_w,), jnp.int32),
            pltpu.VMEM((per_w, d), jnp.float32),
            pltpu.SemaphoreType.DMA,
            pltpu.SemaphoreType.DMA,
        ],
        compiler_params=pltpu.CompilerParams(use_tc_tiling_on_sc=False),
    )
    def gather_kernel(idx_hbm, cache_hbm, out_hbm, idx_v, rows_v, sem, sem_o):
        wid = lax.axis_index("s") * info.num_cores + lax.axis_index("c")
        base = wid * per_w
        b_base = wid * b_per_w
        pltpu.sync_copy(idx_hbm.at[pl.ds(base, per_w)], idx_v)

        rep_off = (wid % jnp.int32(_REPS)) * jnp.int32(t * d)

        def row_body(i, carry):
            t_off = rep_off + (i % jnp.int32(t)) * jnp.int32(d)
            for v in range(_BLK // _LANES):
                sl = pl.ds(i * _BLK + v * _LANES, _LANES)
                idx_v[sl] = idx_v[sl] + t_off
            return carry

        lax.fori_loop(0, n_chunks, row_body, 0)

        copies = []
        for j in range(n_chunks):
            sl = pl.ds(j * _BLK, _BLK)
            copies.append(
                pltpu.async_copy(cache_hbm.at[idx_v.at[sl]], rows_v.at[sl], sem)
            )
        outs = []
        for j in range(n_chunks):
            copies[j].wait()
            outs.append(
                pltpu.async_copy(
                    rows_v.at[pl.ds(j * _BLK, _BLK)],
                    out_hbm.at[pl.ds(b_base + (j // t) * _BLK, _BLK), j % t],
                    sem_o,
                )
            )
        for o in outs:
            o.wait()

    return gather_kernel


def kernel(idx, outputs):
    b, t = idx.shape
    d = outputs.shape[1]
    parts = []
    for p in range(t):
        stride = d**p
        if stride <= d * d:
            parts.append(
                lax.slice(outputs, (0, 0), ((d - 1) * stride + 1, d), (stride, 1))
            )
        else:
            parts.extend(
                lax.slice(outputs, (v * stride, 0), (v * stride + 1, d))
                for v in range(d)
            )
    cache = jnp.tile(jnp.concatenate(parts, axis=0), (_REPS, 1))
    idx_blocks = idx.reshape(b // _BLK, _BLK, t).transpose(0, 2, 1).reshape(-1)
    return _build(b, t, d)(idx_blocks, cache)


# speedup vs baseline: 2.8699x; 1.4208x over previous
import functools

import jax
import jax.numpy as jnp
from jax import lax
from jax.experimental import pallas as pl
from jax.experimental.pallas import tpu as pltpu
from jax.experimental.pallas import tpu_sc as plsc

_LANES = 16
_BLK = 128
_REPS = 32


@functools.cache
def _build(b: int, t: int, d: int):
    total = b * t
    info = plsc.get_sparse_core_info()
    n_workers = info.num_cores * info.num_subcores
    per_w = total // n_workers
    b_per_w = b // n_workers
    n_chunks = per_w // _BLK
    mesh = plsc.VectorSubcoreMesh(core_axis_name="c", subcore_axis_name="s")

    @functools.partial(
        pl.kernel,
        mesh=mesh,
        out_type=jax.ShapeDtypeStruct((b, t, 128), jnp.float32),
        scratch_types=[
            pltpu.VMEM((per_w,), jnp.int32),
            pltpu.VMEM((per_w, d), jnp.float32),
            pltpu.SemaphoreType.DMA,
            pltpu.SemaphoreType.DMA,
        ],
        compiler_params=pltpu.CompilerParams(use_tc_tiling_on_sc=False),
    )
    def gather_kernel(idx_hbm, cache_hbm, out_hbm, idx_v, rows_v, sem, sem_o):
        wid = lax.axis_index("s") * info.num_cores + lax.axis_index("c")
        base = wid * per_w
        b_base = wid * b_per_w
        pltpu.sync_copy(idx_hbm.at[pl.ds(base, per_w)], idx_v)

        rep_off = (wid % jnp.int32(_REPS)) * jnp.int32(t * d)

        def row_body(i, carry):
            t_off = rep_off + (i % jnp.int32(t)) * jnp.int32(d)
            for v in range(_BLK // _LANES):
                sl = pl.ds(i * _BLK + v * _LANES, _LANES)
                idx_v[sl] = idx_v[sl] + t_off
            return carry

        lax.fori_loop(0, n_chunks, row_body, 0)

        copies = []
        for j in range(n_chunks):
            sl = pl.ds(j * _BLK, _BLK)
            copies.append(
                pltpu.async_copy(cache_hbm.at[idx_v.at[sl]], rows_v.at[sl], sem)
            )
        outs = []
        for j in range(n_chunks):
            copies[j].wait()
            outs.append(
                pltpu.async_copy(
                    rows_v.at[pl.ds(j * _BLK, _BLK)],
                    out_hbm.at[
                        pl.ds(b_base + (j // t) * _BLK, _BLK), j % t, pl.ds(0, d)
                    ],
                    sem_o,
                )
            )
        for o in outs:
            o.wait()

    return gather_kernel


def kernel(idx, outputs):
    b, t = idx.shape
    d = outputs.shape[1]
    parts = []
    for p in range(t):
        stride = d**p
        if stride <= d * d:
            parts.append(
                lax.slice(outputs, (0, 0), ((d - 1) * stride + 1, d), (stride, 1))
            )
        else:
            parts.extend(
                lax.slice(outputs, (v * stride, 0), (v * stride + 1, d))
                for v in range(d)
            )
    cache = jnp.tile(jnp.concatenate(parts, axis=0), (_REPS, 1))
    idx_blocks = idx.reshape(b // _BLK, _BLK, t).transpose(0, 2, 1).reshape(-1)
    out_padded = _build(b, t, d)(idx_blocks, cache)
    return lax.slice(out_padded, (0, 0, 0), (b, t, d))
